# baseline (speedup 1.0000x reference)
import functools

import jax
import numpy as np
import jax.numpy as jnp
from jax import lax
from jax.experimental import pallas as pl
from jax.experimental.pallas import tpu as pltpu
from jax.experimental.pallas import tpu_sc as plsc

NC, NS, LANES = 2, 16, 16
CHUNK = 40
ZR = 32


def _mm_body(x_ref, w_ref, o_ref):
    o_ref[...] = jnp.dot(x_ref[...], w_ref[...],
                         preferred_element_type=jnp.float32)


def _precompute(x, wcat):
    n, d = x.shape
    dk = wcat.shape[1]
    rb = 1000
    return pl.pallas_call(
        _mm_body,
        grid=(n // rb,),
        in_specs=[pl.BlockSpec((rb, d), lambda i: (i, 0)),
                  pl.BlockSpec((d, dk), lambda i: (0, 0))],
        out_specs=pl.BlockSpec((rb, dk), lambda i: (i, 0)),
        out_shape=jax.ShapeDtypeStruct((n, dk), jnp.float32),
    )(x, wcat)


def _mesh():
    return plsc.VectorSubcoreMesh(core_axis_name="c", subcore_axis_name="s",
                                  num_cores=NC, num_subcores=NS)


def _npad(n):
    return ((n + NS * ZR - 1) // (NS * ZR)) * (NS * ZR)


def _count_pass(dst_c, npad):
    CH2 = dst_c.shape[1]
    nc2 = dst_c.shape[0] // (NC * NS)
    stripe = npad // NS
    nz = stripe // ZR

    @functools.partial(
        pl.kernel,
        out_type=jax.ShapeDtypeStruct((NC, npad, 128), jnp.float32),
        mesh=_mesh(),
        compiler_params=pltpu.CompilerParams(needs_layout_passes=False),
        scratch_types=[
            pltpu.VMEM((4, CH2), jnp.int32),
            pltpu.VMEM((CH2, 128), jnp.float32),
            pltpu.VMEM((ZR, 128), jnp.float32),
            pltpu.VMEM_SHARED((npad, 128), jnp.float32),
            pltpu.SemaphoreType.DMA,
            pltpu.SemaphoreType.DMA,
            pltpu.SemaphoreType.DMA,
        ],
    )
    def k(dc_hbm, out_hbm, ix2_v, ones_v, zb_v, cnt_sh, ssem, isem0, isem1):
        c = lax.axis_index("c")
        s = lax.axis_index("s")
        jb2 = (c * NS + s) * nc2
        rbase = s * stripe

        @pl.loop(0, ZR)
        def _(kk):
            for j in range(8):
                zb_v[kk, pl.ds(j * LANES, LANES)] = jnp.full(
                    (LANES,), 0.0, jnp.float32)

        @pl.loop(0, CH2)
        def _(kk):
            for j in range(8):
                ones_v[kk, pl.ds(j * LANES, LANES)] = jnp.full(
                    (LANES,), 1.0, jnp.float32)

        @pl.loop(0, nz)
        def _(i):
            r0 = pl.multiple_of(rbase + i * ZR, 8)
            pltpu.sync_copy(zb_v, cnt_sh.at[pl.ds(r0, ZR)])
        plsc.subcore_barrier()

        def wait_cidx(t1):
            @pl.when(lax.rem(t1, 2) == 0)
            def _():
                pltpu.make_async_copy(dc_hbm.at[jb2 + t1],
                                      ix2_v.at[lax.rem(t1, 4)], isem0).wait()

            @pl.when(lax.rem(t1, 2) == 1)
            def _():
                pltpu.make_async_copy(dc_hbm.at[jb2 + t1],
                                      ix2_v.at[lax.rem(t1, 4)], isem1).wait()

        def issue_cidx(t2):
            @pl.when(lax.rem(t2, 2) == 0)
            def _():
                pltpu.async_copy(dc_hbm.at[jb2 + t2],
                                 ix2_v.at[lax.rem(t2, 4)], isem0)

            @pl.when(lax.rem(t2, 2) == 1)
            def _():
                pltpu.async_copy(dc_hbm.at[jb2 + t2],
                                 ix2_v.at[lax.rem(t2, 4)], isem1)

        pltpu.sync_copy(dc_hbm.at[jb2], ix2_v.at[0])
        pltpu.async_copy(dc_hbm.at[jb2 + 1], ix2_v.at[1], isem1)

        @pl.loop(0, nc2)
        def _(t):
            ib = lax.rem(t, 4)

            @pl.when(t + 1 < nc2)
            def _():
                wait_cidx(t + 1)

            @pl.when(t + 2 < nc2)
            def _():
                issue_cidx(t + 2)

            @pl.when(t >= 1)
            def _():
                pltpu.make_async_copy(
                    ones_v, cnt_sh.at[ix2_v.at[ib]], ssem).wait()
            pltpu.async_copy(ones_v, cnt_sh.at[ix2_v.at[ib]], ssem, add=True)

        lastb2 = lax.rem(nc2 - 1, 4)
        pltpu.make_async_copy(
            ones_v, cnt_sh.at[ix2_v.at[lastb2]], ssem).wait()
        plsc.subcore_barrier()

        @pl.loop(0, nz)
        def _(i):
            r0 = pl.multiple_of(rbase + i * ZR, 8)
            pltpu.sync_copy(cnt_sh.at[pl.ds(r0, ZR)],
                            out_hbm.at[c, pl.ds(r0, ZR)])

    return k(dst_c)


def _edge_pass(ab, xv, ei_c, cc16, npad):
    n = ab.shape[0]
    n_chunks_g = ei_c.shape[0]
    n_chunks = n_chunks_g // (NC * NS)
    stripe = npad // NS
    nz = stripe // ZR
    qoffs = [0, 16, 24]

    @functools.partial(
        pl.kernel,
        out_type=jax.ShapeDtypeStruct((NC, npad, 128), jnp.float32),
        mesh=_mesh(),
        compiler_params=pltpu.CompilerParams(needs_layout_passes=False),
        scratch_types=[
            pltpu.VMEM((n,), jnp.float32),
            pltpu.VMEM((LANES,), jnp.float32),
            pltpu.VMEM((4, 2, CHUNK), jnp.int32),
            pltpu.VMEM((2, CHUNK, 128), jnp.int32),
            pltpu.VMEM((2 * CHUNK, 128), jnp.float32),
            pltpu.VMEM((ZR, 128), jnp.float32),
            pltpu.VMEM((CHUNK,), jnp.float32),
            pltpu.VMEM_SHARED((npad, 128), jnp.float32),
            pltpu.SemaphoreType.DMA,
            pltpu.SemaphoreType.DMA,
            pltpu.SemaphoreType.DMA,
            pltpu.SemaphoreType.DMA,
        ],
    )
    def k(ab_hbm, xv_hbm, ei_hbm, cc_hbm, out_hbm,
          xv_v, cc_v, idx_v, rows_v, msg_v, zb_v, q_v,
          agg_sh, gsem, ssem, isem0, isem1):
        c = lax.axis_index("c")
        s = lax.axis_index("s")
        jbase = (c * NS + s) * n_chunks
        rbase = s * stripe

        pltpu.sync_copy(xv_hbm, xv_v)
        pltpu.sync_copy(cc_hbm, cc_v)

        @pl.loop(0, ZR)
        def _(kk):
            for j in range(8):
                zb_v[kk, pl.ds(j * LANES, LANES)] = jnp.full(
                    (LANES,), 0.0, jnp.float32)

        @pl.loop(0, nz)
        def _(i):
            r0 = pl.multiple_of(rbase + i * ZR, 8)
            pltpu.sync_copy(zb_v, agg_sh.at[pl.ds(r0, ZR)])
        plsc.subcore_barrier()

        ccv = cc_v[...]

        def wait_idx(t1):
            @pl.when(lax.rem(t1, 2) == 0)
            def _():
                pltpu.make_async_copy(ei_hbm.at[jbase + t1],
                                      idx_v.at[lax.rem(t1, 4)], isem0).wait()

            @pl.when(lax.rem(t1, 2) == 1)
            def _():
                pltpu.make_async_copy(ei_hbm.at[jbase + t1],
                                      idx_v.at[lax.rem(t1, 4)], isem1).wait()

        def issue_idx(t2):
            @pl.when(lax.rem(t2, 2) == 0)
            def _():
                pltpu.async_copy(ei_hbm.at[jbase + t2],
                                 idx_v.at[lax.rem(t2, 4)], isem0)

            @pl.when(lax.rem(t2, 2) == 1)
            def _():
                pltpu.async_copy(ei_hbm.at[jbase + t2],
                                 idx_v.at[lax.rem(t2, 4)], isem1)

        pltpu.sync_copy(ei_hbm.at[jbase], idx_v.at[0])
        pltpu.async_copy(ab_hbm.at[idx_v.at[0, 0]], rows_v.at[0], gsem)
        pltpu.async_copy(ei_hbm.at[jbase + 1], idx_v.at[1], isem1)

        @pl.loop(0, n_chunks)
        def _(t):
            p = lax.rem(t, 2)
            np_ = 1 - p
            ib = lax.rem(t, 4)
            ib1 = lax.rem(t + 1, 4)

            @pl.when(t + 1 < n_chunks)
            def _():
                wait_idx(t + 1)

            pltpu.make_async_copy(
                ab_hbm.at[idx_v.at[ib, 0]], rows_v.at[p], gsem).wait()

            @pl.when(t + 1 < n_chunks)
            def _():
                pltpu.async_copy(
                    ab_hbm.at[idx_v.at[ib1, 0]], rows_v.at[np_], gsem)

            @pl.when(t + 2 < n_chunks)
            def _():
                issue_idx(t + 2)

            for off in qoffs:
                sl = pl.ds(off, LANES)
                xs = plsc.load_gather(xv_v, [idx_v[ib, 0, sl]])
                xd = plsc.load_gather(xv_v, [idx_v[ib, 1, sl]])
                z = xs - xd + ccv
                q_v[sl] = 1.0 / (1.0 + jnp.exp(-z))

            mrow = pl.multiple_of(p * CHUNK, 8)
            mref = msg_v.at[pl.ds(mrow, CHUNK)]

            @pl.when(t >= 1)
            def _():
                pltpu.make_async_copy(
                    mref, agg_sh.at[idx_v.at[ib, 1]], ssem).wait()

            @plsc.parallel_loop(0, CHUNK, unroll=4)
            def _(kk):
                qs = plsc.load_gather(q_v, [jnp.full((LANES,), kk, jnp.int32)])
                kr = p * CHUNK + kk
                for j in range(4):
                    wa = rows_v[p, kk, pl.ds(j * 16, 16)]
                    wb = rows_v[p, kk, pl.ds(64 + j * 16, 16)]
                    a32 = plsc.bitcast(wa, jnp.bfloat16)
                    b32 = plsc.bitcast(wb, jnp.bfloat16)
                    ae, ao = plsc.unpack(a32, format=plsc.PackFormat.INTERLEAVED)
                    be, bo = plsc.unpack(b32, format=plsc.PackFormat.INTERLEAVED)
                    msg_v[kr, pl.ds(j * 16, 16)] = be + qs * ae
                    msg_v[kr, pl.ds(64 + j * 16, 16)] = bo + qs * ao

            pltpu.async_copy(mref, agg_sh.at[idx_v.at[ib, 1]], ssem,
                             add=True)

        lastp = pl.multiple_of(lax.rem(n_chunks - 1, 2) * CHUNK, 8)
        lastb = lax.rem(n_chunks - 1, 4)
        pltpu.make_async_copy(
            msg_v.at[pl.ds(lastp, CHUNK)], agg_sh.at[idx_v.at[lastb, 1]],
            ssem).wait()
        plsc.subcore_barrier()

        @pl.loop(0, nz)
        def _(i):
            r0 = pl.multiple_of(rbase + i * ZR, 8)
            pltpu.sync_copy(agg_sh.at[pl.ds(r0, ZR)],
                            out_hbm.at[c, pl.ds(r0, ZR)])

    return k(ab, xv, ei_c, cc16)


def _post_mm_body(p0_ref, p1_ref, c0_ref, c1_ref, ab_ref, w_ref, b_ref,
                  s_ref, o_ref):
    s0 = s_ref[0]
    agg = (p0_ref[...] + p1_ref[...]
           + ab_ref[:, 128:] + s0 * ab_ref[:, :128])
    cnt = c0_ref[:, :1] + c1_ref[:, :1] + 1.0
    h = jnp.maximum(agg / cnt + b_ref[...], 0.0)
    o_ref[...] = jnp.dot(h, w_ref[...], preferred_element_type=jnp.float32)


def _finish_mm(p0, p1, c0, c1, ab, wcat2, b2d, s0arr):
    n = ab.shape[0]
    dk = wcat2.shape[1]
    rb = 1000
    return pl.pallas_call(
        _post_mm_body,
        grid=(n // rb,),
        in_specs=[
            pl.BlockSpec((rb, 128), lambda i: (i, 0)),
            pl.BlockSpec((rb, 128), lambda i: (i, 0)),
            pl.BlockSpec((rb, 128), lambda i: (i, 0)),
            pl.BlockSpec((rb, 128), lambda i: (i, 0)),
            pl.BlockSpec((rb, 256), lambda i: (i, 0)),
            pl.BlockSpec((128, dk), lambda i: (0, 0)),
            pl.BlockSpec((1, 128), lambda i: (0, 0)),
            pl.BlockSpec(memory_space=pltpu.SMEM),
        ],
        out_specs=pl.BlockSpec((rb, dk), lambda i: (i, 0)),
        out_shape=jax.ShapeDtypeStruct((n, dk), jnp.float32),
    )(p0, p1, c0, c1, ab, wcat2, b2d, s0arr)


def _post_body(p0_ref, p1_ref, c0_ref, c1_ref, ab_ref, x_ref, b_ref, s_ref,
               o_ref, *, residual):
    s0 = s_ref[0]
    agg = (p0_ref[...] + p1_ref[...]
           + ab_ref[:, 128:] + s0 * ab_ref[:, :128])
    cnt = c0_ref[:, :1] + c1_ref[:, :1] + 1.0
    h = agg / cnt + b_ref[...]
    if residual:
        h = h + x_ref[...]
    o_ref[...] = jnp.maximum(h, 0.0)


def _finish(p0, p1, c0, c1, ab, xres, b2d, s0arr, residual):
    n = ab.shape[0]
    rb = 1000
    return pl.pallas_call(
        functools.partial(_post_body, residual=residual),
        grid=(n // rb,),
        in_specs=[
            pl.BlockSpec((rb, 128), lambda i: (i, 0)),
            pl.BlockSpec((rb, 128), lambda i: (i, 0)),
            pl.BlockSpec((rb, 128), lambda i: (i, 0)),
            pl.BlockSpec((rb, 128), lambda i: (i, 0)),
            pl.BlockSpec((rb, 256), lambda i: (i, 0)),
            pl.BlockSpec((rb, 128), lambda i: (i, 0)),
            pl.BlockSpec((1, 128), lambda i: (0, 0)),
            pl.BlockSpec(memory_space=pltpu.SMEM),
        ],
        out_specs=pl.BlockSpec((rb, 128), lambda i: (i, 0)),
        out_shape=jax.ShapeDtypeStruct((n, 128), jnp.float32),
    )(p0, p1, c0, c1, ab, xres, b2d, s0arr)


def kernel(x, edge_index, u1, c1, W1, b1, u2, c2, W2, b2):
    n, d = x.shape
    e = edge_index.shape[1]
    ei_c = edge_index.reshape(2, e // CHUNK, CHUNK).transpose(1, 0, 2)
    dst_c = edge_index[1].reshape(e // (2 * CHUNK), 2 * CHUNK)
    npad = _npad(n)

    pos = np.arange(128)
    srccol = 16 * (pos // 32) + (pos % 32) // 2 + 64 * (pos % 2)
    colperm = np.concatenate([srccol, 128 + srccol])

    def ab_table(pre):
        abbf = pre[:, :256][:, colperm].astype(jnp.bfloat16)
        return jax.lax.bitcast_convert_type(
            abbf.reshape(n, 128, 2), jnp.int32)

    cnt = _count_pass(dst_c, npad)
    cnt0 = cnt[0, :n]
    cnt1 = cnt[1, :n]

    def wcat_of(u, W):
        wa = W[:, :d] - W[:, d:]
        wb = W[:, d:]
        uv = (u[:, 0] - u[:, 1])[:, None]
        return jnp.concatenate(
            [wa, wb, jnp.broadcast_to(uv, (d, 128))], axis=1)

    wcat1 = wcat_of(u1, W1)
    wcat2 = wcat_of(u2, W2)
    cc1 = c1[0] - c1[1]
    cc2 = c2[0] - c2[1]

    pre1 = _precompute(x, wcat1)
    ab1 = pre1[:, :256]
    xv1 = pre1[:, 256]
    agg1 = _edge_pass(ab_table(pre1), xv1, ei_c,
                      jnp.full((16,), cc1, jnp.float32), npad)
    pre2 = _finish_mm(agg1[0, :n], agg1[1, :n], cnt0, cnt1, ab1, wcat2,
                      b1.reshape(1, d), jax.nn.sigmoid(cc1)[None])
    ab2 = pre2[:, :256]
    xv2 = pre2[:, 256]
    agg2 = _edge_pass(ab_table(pre2), xv2, ei_c,
                      jnp.full((16,), cc2, jnp.float32), npad)
    out = _finish(agg2[0, :n], agg2[1, :n], cnt0, cnt1, ab2, x,
                  b2.reshape(1, d), jax.nn.sigmoid(cc2)[None], residual=True)
    return (out, edge_index)


# speedup vs baseline: 1.1334x; 1.1334x over previous
import functools

import jax
import jax.numpy as jnp
from jax import lax
from jax.experimental import pallas as pl
from jax.experimental.pallas import tpu as pltpu
from jax.experimental.pallas import tpu_sc as plsc

NC, NS, LANES = 2, 16, 16
CHUNK = 40
ZR = 32


def _mm_body(x_ref, w_ref, o_ref):
    o_ref[...] = jnp.dot(x_ref[...], w_ref[...],
                         preferred_element_type=jnp.float32)


def _precompute(x, wcat):
    n, d = x.shape
    dk = wcat.shape[1]
    rb = 1000
    return pl.pallas_call(
        _mm_body,
        grid=(n // rb,),
        in_specs=[pl.BlockSpec((rb, d), lambda i: (i, 0)),
                  pl.BlockSpec((d, dk), lambda i: (0, 0))],
        out_specs=pl.BlockSpec((rb, dk), lambda i: (i, 0)),
        out_shape=jax.ShapeDtypeStruct((n, dk), jnp.float32),
    )(x, wcat)


def _mesh():
    return plsc.VectorSubcoreMesh(core_axis_name="c", subcore_axis_name="s",
                                  num_cores=NC, num_subcores=NS)


def _npad(n):
    return ((n + NS * ZR - 1) // (NS * ZR)) * (NS * ZR)


def _count_pass(dst_c, npad):
    CH2 = dst_c.shape[1]
    nc2 = dst_c.shape[0] // (NC * NS)

    @functools.partial(
        pl.kernel,
        out_type=jax.ShapeDtypeStruct((NC * NS, npad), jnp.float32),
        mesh=_mesh(),
        compiler_params=pltpu.CompilerParams(needs_layout_passes=False),
        scratch_types=[
            pltpu.VMEM((4, CH2), jnp.int32),
            pltpu.VMEM((npad,), jnp.float32),
            pltpu.SemaphoreType.DMA,
            pltpu.SemaphoreType.DMA,
        ],
    )
    def k(dc_hbm, out_hbm, ix2_v, hist_v, isem0, isem1):
        c = lax.axis_index("c")
        s = lax.axis_index("s")
        w = c * NS + s
        jb2 = w * nc2

        @pl.loop(0, npad // LANES)
        def _(kk):
            hist_v[pl.ds(kk * LANES, LANES)] = jnp.full(
                (LANES,), 0.0, jnp.float32)

        def wait_cidx(t1):
            @pl.when(lax.rem(t1, 2) == 0)
            def _():
                pltpu.make_async_copy(dc_hbm.at[jb2 + t1],
                                      ix2_v.at[lax.rem(t1, 4)], isem0).wait()

            @pl.when(lax.rem(t1, 2) == 1)
            def _():
                pltpu.make_async_copy(dc_hbm.at[jb2 + t1],
                                      ix2_v.at[lax.rem(t1, 4)], isem1).wait()

        def issue_cidx(t2):
            @pl.when(lax.rem(t2, 2) == 0)
            def _():
                pltpu.async_copy(dc_hbm.at[jb2 + t2],
                                 ix2_v.at[lax.rem(t2, 4)], isem0)

            @pl.when(lax.rem(t2, 2) == 1)
            def _():
                pltpu.async_copy(dc_hbm.at[jb2 + t2],
                                 ix2_v.at[lax.rem(t2, 4)], isem1)

        pltpu.sync_copy(dc_hbm.at[jb2], ix2_v.at[0])
        pltpu.async_copy(dc_hbm.at[jb2 + 1], ix2_v.at[1], isem1)

        @pl.loop(0, nc2)
        def _(t):
            ib = lax.rem(t, 4)

            @pl.when(t + 1 < nc2)
            def _():
                wait_cidx(t + 1)

            @pl.when(t + 2 < nc2)
            def _():
                issue_cidx(t + 2)

            for g in range(CH2 // LANES):
                iv = ix2_v[ib, pl.ds(g * LANES, LANES)]
                plsc.addupdate_scatter(
                    hist_v, [iv], jnp.full((LANES,), 1.0, jnp.float32))

        pltpu.sync_copy(hist_v, out_hbm.at[w])

    return k(dst_c)


def _edge_pass(ab, xv, ei_c, cc16, npad):
    n = ab.shape[0]
    n_chunks_g = ei_c.shape[0]
    n_chunks = n_chunks_g // (NC * NS)
    stripe = npad // NS
    nz = stripe // ZR
    qoffs = [0, 16, 24]

    @functools.partial(
        pl.kernel,
        out_type=jax.ShapeDtypeStruct((NC, npad, 128), jnp.float32),
        mesh=_mesh(),
        compiler_params=pltpu.CompilerParams(needs_layout_passes=False),
        scratch_types=[
            pltpu.VMEM((n,), jnp.float32),
            pltpu.VMEM((LANES,), jnp.float32),
            pltpu.VMEM((4, 2, CHUNK), jnp.int32),
            pltpu.VMEM((2, CHUNK, 256), jnp.float32),
            pltpu.VMEM((2 * CHUNK, 128), jnp.float32),
            pltpu.VMEM((ZR, 128), jnp.float32),
            pltpu.VMEM((CHUNK,), jnp.float32),
            pltpu.VMEM_SHARED((npad, 128), jnp.float32),
            pltpu.SemaphoreType.DMA,
            pltpu.SemaphoreType.DMA,
            pltpu.SemaphoreType.DMA,
            pltpu.SemaphoreType.DMA,
        ],
    )
    def k(ab_hbm, xv_hbm, ei_hbm, cc_hbm, out_hbm,
          xv_v, cc_v, idx_v, rows_v, msg_v, zb_v, q_v,
          agg_sh, gsem, ssem, isem0, isem1):
        c = lax.axis_index("c")
        s = lax.axis_index("s")
        jbase = (c * NS + s) * n_chunks
        rbase = s * stripe

        pltpu.sync_copy(xv_hbm, xv_v)
        pltpu.sync_copy(cc_hbm, cc_v)

        @pl.loop(0, ZR)
        def _(kk):
            for j in range(8):
                zb_v[kk, pl.ds(j * LANES, LANES)] = jnp.full(
                    (LANES,), 0.0, jnp.float32)

        @pl.loop(0, nz)
        def _(i):
            r0 = pl.multiple_of(rbase + i * ZR, 8)
            pltpu.sync_copy(zb_v, agg_sh.at[pl.ds(r0, ZR)])
        plsc.subcore_barrier()

        ccv = cc_v[...]

        def wait_idx(t1):
            @pl.when(lax.rem(t1, 2) == 0)
            def _():
                pltpu.make_async_copy(ei_hbm.at[jbase + t1],
                                      idx_v.at[lax.rem(t1, 4)], isem0).wait()

            @pl.when(lax.rem(t1, 2) == 1)
            def _():
                pltpu.make_async_copy(ei_hbm.at[jbase + t1],
                                      idx_v.at[lax.rem(t1, 4)], isem1).wait()

        def issue_idx(t2):
            @pl.when(lax.rem(t2, 2) == 0)
            def _():
                pltpu.async_copy(ei_hbm.at[jbase + t2],
                                 idx_v.at[lax.rem(t2, 4)], isem0)

            @pl.when(lax.rem(t2, 2) == 1)
            def _():
                pltpu.async_copy(ei_hbm.at[jbase + t2],
                                 idx_v.at[lax.rem(t2, 4)], isem1)

        pltpu.sync_copy(ei_hbm.at[jbase], idx_v.at[0])
        pltpu.async_copy(ab_hbm.at[idx_v.at[0, 0]], rows_v.at[0], gsem)
        pltpu.async_copy(ei_hbm.at[jbase + 1], idx_v.at[1], isem1)

        @pl.loop(0, n_chunks)
        def _(t):
            p = lax.rem(t, 2)
            np_ = 1 - p
            ib = lax.rem(t, 4)
            ib1 = lax.rem(t + 1, 4)

            @pl.when(t + 1 < n_chunks)
            def _():
                wait_idx(t + 1)

            pltpu.make_async_copy(
                ab_hbm.at[idx_v.at[ib, 0]], rows_v.at[p], gsem).wait()

            @pl.when(t + 1 < n_chunks)
            def _():
                pltpu.async_copy(
                    ab_hbm.at[idx_v.at[ib1, 0]], rows_v.at[np_], gsem)

            @pl.when(t + 2 < n_chunks)
            def _():
                issue_idx(t + 2)

            for off in qoffs:
                sl = pl.ds(off, LANES)
                xs = plsc.load_gather(xv_v, [idx_v[ib, 0, sl]])
                xd = plsc.load_gather(xv_v, [idx_v[ib, 1, sl]])
                z = xs - xd + ccv
                q_v[sl] = 1.0 / (1.0 + jnp.exp(-z))

            mrow = pl.multiple_of(p * CHUNK, 8)
            mref = msg_v.at[pl.ds(mrow, CHUNK)]

            @pl.when(t >= 1)
            def _():
                pltpu.make_async_copy(
                    mref, agg_sh.at[idx_v.at[ib, 1]], ssem).wait()

            @plsc.parallel_loop(0, CHUNK, unroll=4)
            def _(kk):
                qs = plsc.load_gather(q_v, [jnp.full((LANES,), kk, jnp.int32)])
                kr = p * CHUNK + kk
                for j in range(8):
                    sl = pl.ds(j * LANES, LANES)
                    slb = pl.ds(128 + j * LANES, LANES)
                    msg_v[kr, sl] = rows_v[p, kk, slb] + qs * rows_v[p, kk, sl]

            pltpu.async_copy(mref, agg_sh.at[idx_v.at[ib, 1]], ssem,
                             add=True)

        lastp = pl.multiple_of(lax.rem(n_chunks - 1, 2) * CHUNK, 8)
        lastb = lax.rem(n_chunks - 1, 4)
        pltpu.make_async_copy(
            msg_v.at[pl.ds(lastp, CHUNK)], agg_sh.at[idx_v.at[lastb, 1]],
            ssem).wait()
        plsc.subcore_barrier()

        @pl.loop(0, nz)
        def _(i):
            r0 = pl.multiple_of(rbase + i * ZR, 8)
            pltpu.sync_copy(agg_sh.at[pl.ds(r0, ZR)],
                            out_hbm.at[c, pl.ds(r0, ZR)])

    return k(ab, xv, ei_c, cc16)


def _post_mm_body(p0_ref, p1_ref, cn_ref, ab_ref, w_ref, b_ref,
                  s_ref, o_ref):
    s0 = s_ref[0]
    agg = (p0_ref[...] + p1_ref[...]
           + ab_ref[:, 128:] + s0 * ab_ref[:, :128])
    cnt = jnp.sum(cn_ref[...], axis=1)[:, None] + 1.0
    h = jnp.maximum(agg / cnt + b_ref[...], 0.0)
    o_ref[...] = jnp.dot(h, w_ref[...], preferred_element_type=jnp.float32)


def _finish_mm(p0, p1, cn, ab, wcat2, b2d, s0arr):
    n = ab.shape[0]
    dk = wcat2.shape[1]
    rb = 1000
    return pl.pallas_call(
        _post_mm_body,
        grid=(n // rb,),
        in_specs=[
            pl.BlockSpec((rb, 128), lambda i: (i, 0)),
            pl.BlockSpec((rb, 128), lambda i: (i, 0)),
            pl.BlockSpec((rb, 32), lambda i: (i, 0)),
            pl.BlockSpec((rb, 256), lambda i: (i, 0)),
            pl.BlockSpec((128, dk), lambda i: (0, 0)),
            pl.BlockSpec((1, 128), lambda i: (0, 0)),
            pl.BlockSpec(memory_space=pltpu.SMEM),
        ],
        out_specs=pl.BlockSpec((rb, dk), lambda i: (i, 0)),
        out_shape=jax.ShapeDtypeStruct((n, dk), jnp.float32),
    )(p0, p1, cn, ab, wcat2, b2d, s0arr)


def _post_body(p0_ref, p1_ref, cn_ref, ab_ref, x_ref, b_ref, s_ref,
               o_ref, *, residual):
    s0 = s_ref[0]
    agg = (p0_ref[...] + p1_ref[...]
           + ab_ref[:, 128:] + s0 * ab_ref[:, :128])
    cnt = jnp.sum(cn_ref[...], axis=1)[:, None] + 1.0
    h = agg / cnt + b_ref[...]
    if residual:
        h = h + x_ref[...]
    o_ref[...] = jnp.maximum(h, 0.0)


def _finish(p0, p1, cn, ab, xres, b2d, s0arr, residual):
    n = ab.shape[0]
    rb = 1000
    return pl.pallas_call(
        functools.partial(_post_body, residual=residual),
        grid=(n // rb,),
        in_specs=[
            pl.BlockSpec((rb, 128), lambda i: (i, 0)),
            pl.BlockSpec((rb, 128), lambda i: (i, 0)),
            pl.BlockSpec((rb, 32), lambda i: (i, 0)),
            pl.BlockSpec((rb, 256), lambda i: (i, 0)),
            pl.BlockSpec((rb, 128), lambda i: (i, 0)),
            pl.BlockSpec((1, 128), lambda i: (0, 0)),
            pl.BlockSpec(memory_space=pltpu.SMEM),
        ],
        out_specs=pl.BlockSpec((rb, 128), lambda i: (i, 0)),
        out_shape=jax.ShapeDtypeStruct((n, 128), jnp.float32),
    )(p0, p1, cn, ab, xres, b2d, s0arr)


def kernel(x, edge_index, u1, c1, W1, b1, u2, c2, W2, b2):
    n, d = x.shape
    e = edge_index.shape[1]
    ei_c = edge_index.reshape(2, e // CHUNK, CHUNK).transpose(1, 0, 2)
    dst_c = edge_index[1].reshape(e // (2 * CHUNK), 2 * CHUNK)
    npad = _npad(n)

    cnt = _count_pass(dst_c, npad)[:, :n].T

    def wcat_of(u, W):
        wa = W[:, :d] - W[:, d:]
        wb = W[:, d:]
        uv = (u[:, 0] - u[:, 1])[:, None]
        return jnp.concatenate(
            [wa, wb, jnp.broadcast_to(uv, (d, 128))], axis=1)

    wcat1 = wcat_of(u1, W1)
    wcat2 = wcat_of(u2, W2)
    cc1 = c1[0] - c1[1]
    cc2 = c2[0] - c2[1]

    pre1 = _precompute(x, wcat1)
    ab1 = pre1[:, :256]
    xv1 = pre1[:, 256]
    agg1 = _edge_pass(ab1, xv1, ei_c, jnp.full((16,), cc1, jnp.float32), npad)
    pre2 = _finish_mm(agg1[0, :n], agg1[1, :n], cnt, ab1, wcat2,
                      b1.reshape(1, d), jax.nn.sigmoid(cc1)[None])
    ab2 = pre2[:, :256]
    xv2 = pre2[:, 256]
    agg2 = _edge_pass(ab2, xv2, ei_c, jnp.full((16,), cc2, jnp.float32), npad)
    out = _finish(agg2[0, :n], agg2[1, :n], cnt, ab2, x,
                  b2.reshape(1, d), jax.nn.sigmoid(cc2)[None], residual=True)
    return (out, edge_index)
